# Initial kernel scaffold; baseline (speedup 1.0000x reference)
import jax
import jax.numpy as jnp
from jax.experimental import pallas as pl


def kernel(x_bd):
    raise NotImplementedError("write your pallas kernel here")


# speedup vs baseline: 1.0748x; 1.0748x over previous
import jax
import jax.numpy as jnp
from jax.experimental import pallas as pl
from jax.experimental.pallas import tpu as pltpu

_D = 512
_TOK_BLK = 1024


def _body(x_ref, o_ref):
    x = x_ref[...]
    t = x.shape[0]

    mark = x[:, 0:1] > 0.5
    shl = x[:, 1:2] > 0.5
    shr = jnp.logical_and(jnp.logical_not(shl), x[:, 2:3] > 0.5)
    active = jnp.logical_and(mark, jnp.logical_or(shl, shr))

    def first_set(sl):
        iota = jax.lax.broadcasted_iota(jnp.int32, (t, 16), 1)
        masked = jnp.where(sl > 0.5, iota, 16)
        m = jnp.min(masked, axis=1, keepdims=True)
        return jnp.where(m == 16, 0, m)

    lo = first_set(x[:, 16:32])
    hi = first_set(x[:, 32:48])
    sa = first_set(x[:, 48:64])

    value = lo + 16 * hi
    shl_res = jnp.bitwise_and(jnp.left_shift(value, sa), 255)
    shr_res = jnp.right_shift(value, sa)
    res = jnp.where(shl, shl_res, shr_res)
    res_lo = 96 + jnp.bitwise_and(res, 15)
    res_hi = 112 + jnp.right_shift(res, 4)

    col = jax.lax.broadcasted_iota(jnp.int32, (t, _D), 1)
    hit = jnp.logical_or(col == res_lo, col == res_hi)
    add = jnp.where(jnp.logical_and(active, hit), 2.0, 0.0).astype(x.dtype)
    o_ref[...] = x + add


def kernel(x_bd):
    b, s, d = x_bd.shape
    n = b * s
    x2 = x_bd.reshape(n, d)
    out = pl.pallas_call(
        _body,
        grid=(n // _TOK_BLK,),
        in_specs=[pl.BlockSpec((_TOK_BLK, d), lambda i: (i, 0))],
        out_specs=pl.BlockSpec((_TOK_BLK, d), lambda i: (i, 0)),
        out_shape=jax.ShapeDtypeStruct((n, d), x_bd.dtype),
    )(x2)
    return out.reshape(b, s, d)


# speedup vs baseline: 1.2904x; 1.2006x over previous
import jax
import jax.numpy as jnp
from jax.experimental import pallas as pl
from jax.experimental.pallas import tpu as pltpu

_D = 512
_TOK_BLK = 2048


def _body(x_ref, o_ref):
    x = x_ref[...]
    t = x.shape[0]

    mark = x[:, 0:1] > 0.5
    shl = x[:, 1:2] > 0.5
    shr = jnp.logical_and(jnp.logical_not(shl), x[:, 2:3] > 0.5)
    active = jnp.logical_and(mark, jnp.logical_or(shl, shr))

    def first_set(sl):
        iota = jax.lax.broadcasted_iota(jnp.int32, (t, 16), 1)
        masked = jnp.where(sl > 0.5, iota, 16)
        m = jnp.min(masked, axis=1, keepdims=True)
        return jnp.where(m == 16, 0, m)

    lo = first_set(x[:, 16:32])
    hi = first_set(x[:, 32:48])
    sa = first_set(x[:, 48:64])

    value = lo + 16 * hi
    shl_res = jnp.bitwise_and(jnp.left_shift(value, sa), 255)
    shr_res = jnp.right_shift(value, sa)
    res = jnp.where(shl, shl_res, shr_res)
    res_lo = 96 + jnp.bitwise_and(res, 15)
    res_hi = 112 + jnp.right_shift(res, 4)

    col = jax.lax.broadcasted_iota(jnp.int32, (t, _D), 1)
    hit = jnp.logical_or(col == res_lo, col == res_hi)
    add = jnp.where(jnp.logical_and(active, hit), 2.0, 0.0).astype(x.dtype)
    o_ref[...] = x + add


def kernel(x_bd):
    b, s, d = x_bd.shape
    n = b * s
    x2 = x_bd.reshape(n, d)
    out = pl.pallas_call(
        _body,
        grid=(n // _TOK_BLK,),
        in_specs=[pl.BlockSpec((_TOK_BLK, d), lambda i: (i, 0))],
        out_specs=pl.BlockSpec((_TOK_BLK, d), lambda i: (i, 0)),
        out_shape=jax.ShapeDtypeStruct((n, d), x_bd.dtype),
    )(x2)
    return out.reshape(b, s, d)


# speedup vs baseline: 1.4072x; 1.0905x over previous
import jax
import jax.numpy as jnp
from jax.experimental import pallas as pl
from jax.experimental.pallas import tpu as pltpu

_D = 512
_TOK_BLK = 4096


def _body(x_ref, o_ref):
    x = x_ref[...]
    t = x.shape[0]

    mark = x[:, 0:1] > 0.5
    shl = x[:, 1:2] > 0.5
    shr = jnp.logical_and(jnp.logical_not(shl), x[:, 2:3] > 0.5)
    active = jnp.logical_and(mark, jnp.logical_or(shl, shr))

    def first_set(sl):
        iota = jax.lax.broadcasted_iota(jnp.int32, (t, 16), 1)
        masked = jnp.where(sl > 0.5, iota, 16)
        m = jnp.min(masked, axis=1, keepdims=True)
        return jnp.where(m == 16, 0, m)

    lo = first_set(x[:, 16:32])
    hi = first_set(x[:, 32:48])
    sa = first_set(x[:, 48:64])

    value = lo + 16 * hi
    shl_res = jnp.bitwise_and(jnp.left_shift(value, sa), 255)
    shr_res = jnp.right_shift(value, sa)
    res = jnp.where(shl, shl_res, shr_res)
    res_lo = 96 + jnp.bitwise_and(res, 15)
    res_hi = 112 + jnp.right_shift(res, 4)

    col = jax.lax.broadcasted_iota(jnp.int32, (t, _D), 1)
    hit = jnp.logical_or(col == res_lo, col == res_hi)
    add = jnp.where(jnp.logical_and(active, hit), 2.0, 0.0).astype(x.dtype)
    o_ref[...] = x + add


def kernel(x_bd):
    b, s, d = x_bd.shape
    n = b * s
    x2 = x_bd.reshape(n, d)
    out = pl.pallas_call(
        _body,
        grid=(n // _TOK_BLK,),
        in_specs=[pl.BlockSpec((_TOK_BLK, d), lambda i: (i, 0))],
        out_specs=pl.BlockSpec((_TOK_BLK, d), lambda i: (i, 0)),
        out_shape=jax.ShapeDtypeStruct((n, d), x_bd.dtype),
    )(x2)
    return out.reshape(b, s, d)


# speedup vs baseline: 1.4772x; 1.0497x over previous
import jax
import jax.numpy as jnp
from jax.experimental import pallas as pl
from jax.experimental.pallas import tpu as pltpu

_D = 512
_TOK_BLK = 4096


def _body(x_ref, o_ref):
    x = x_ref[...]
    t = x.shape[0]

    mark = x[:, 0:1] > 0.5
    shl = x[:, 1:2] > 0.5
    shr = jnp.logical_and(jnp.logical_not(shl), x[:, 2:3] > 0.5)
    active = jnp.logical_and(mark, jnp.logical_or(shl, shr))

    def first_set(sl):
        iota = jax.lax.broadcasted_iota(jnp.int32, (t, 16), 1)
        masked = jnp.where(sl > 0.5, iota, 16)
        m = jnp.min(masked, axis=1, keepdims=True)
        return jnp.where(m == 16, 0, m)

    lo = first_set(x[:, 16:32])
    hi = first_set(x[:, 32:48])
    sa = first_set(x[:, 48:64])

    value = lo + 16 * hi
    shl_res = jnp.bitwise_and(jnp.left_shift(value, sa), 255)
    shr_res = jnp.right_shift(value, sa)
    res = jnp.where(shl, shl_res, shr_res)
    res_lo = jnp.bitwise_and(res, 15)
    res_hi = 16 + jnp.right_shift(res, 4)

    col = jax.lax.broadcasted_iota(jnp.int32, (t, 32), 1)
    hit = jnp.logical_or(col == res_lo, col == res_hi)
    add = jnp.where(jnp.logical_and(active, hit), 2.0, 0.0).astype(x.dtype)
    o_ref[:, 0:96] = x[:, 0:96]
    o_ref[:, 96:128] = x[:, 96:128] + add
    o_ref[:, 128:512] = x[:, 128:512]


def kernel(x_bd):
    b, s, d = x_bd.shape
    n = b * s
    x2 = x_bd.reshape(n, d)
    out = pl.pallas_call(
        _body,
        grid=(n // _TOK_BLK,),
        in_specs=[pl.BlockSpec((_TOK_BLK, d), lambda i: (i, 0))],
        out_specs=pl.BlockSpec((_TOK_BLK, d), lambda i: (i, 0)),
        out_shape=jax.ShapeDtypeStruct((n, d), x_bd.dtype),
    )(x2)
    return out.reshape(b, s, d)


# speedup vs baseline: 2.0603x; 1.3947x over previous
import jax
import jax.numpy as jnp
import numpy as np
from jax.experimental import pallas as pl
from jax.experimental.pallas import tpu as pltpu

_D = 512
_TOK_BLK = 4096

_W = np.zeros((64, 128), np.float32)
_W[0:3, 0] = [1.0, 2.0, 4.0]
for j, base in enumerate((16, 32, 48)):
    _W[base:base + 16, 1 + j] = [float(1 << k) for k in range(16)]
_W = jnp.asarray(_W, dtype=jnp.bfloat16)


def _body(x_ref, w_ref, o_ref):
    x = x_ref[...]
    t = x.shape[0]

    bits = (x[:, 0:64] > 0.5).astype(jnp.bfloat16)
    m = jnp.dot(bits, w_ref[...], preferred_element_type=jnp.float32)
    mi = m.astype(jnp.int32)

    flags = mi[:, 0:1]
    mark = jnp.bitwise_and(flags, 1) > 0
    shl = jnp.bitwise_and(flags, 2) > 0
    shr = jnp.logical_and(jnp.logical_not(shl), jnp.bitwise_and(flags, 4) > 0)
    active = jnp.logical_and(mark, jnp.logical_or(shl, shr))

    def first_set(col):
        v = mi[:, col:col + 1]
        low = jnp.bitwise_and(v, -v)
        f = low.astype(jnp.float32)
        e = jnp.right_shift(jax.lax.bitcast_convert_type(f, jnp.int32), 23) - 127
        return jnp.where(v == 0, 0, e)

    lo = first_set(1)
    hi = first_set(2)
    sa = first_set(3)

    value = lo + 16 * hi
    shl_res = jnp.bitwise_and(jnp.left_shift(value, sa), 255)
    shr_res = jnp.right_shift(value, sa)
    res = jnp.where(shl, shl_res, shr_res)
    res_lo = jnp.bitwise_and(res, 15)
    res_hi = 16 + jnp.right_shift(res, 4)

    col = jax.lax.broadcasted_iota(jnp.int32, (t, 32), 1)
    hit = jnp.logical_or(col == res_lo, col == res_hi)
    add = jnp.where(jnp.logical_and(active, hit), 2.0, 0.0).astype(x.dtype)
    o_ref[:, 0:96] = x[:, 0:96]
    o_ref[:, 96:128] = x[:, 96:128] + add
    o_ref[:, 128:512] = x[:, 128:512]


def kernel(x_bd):
    b, s, d = x_bd.shape
    n = b * s
    x2 = x_bd.reshape(n, d)
    out = pl.pallas_call(
        _body,
        grid=(n // _TOK_BLK,),
        in_specs=[
            pl.BlockSpec((_TOK_BLK, d), lambda i: (i, 0)),
            pl.BlockSpec((64, 128), lambda i: (0, 0)),
        ],
        out_specs=pl.BlockSpec((_TOK_BLK, d), lambda i: (i, 0)),
        out_shape=jax.ShapeDtypeStruct((n, d), x_bd.dtype),
    )(x2, _W)
    return out.reshape(b, s, d)
